# Initial kernel scaffold; baseline (speedup 1.0000x reference)
import jax
import jax.numpy as jnp
from jax.experimental import pallas as pl


def kernel(idx, p_pos, p_ori, a, gaussian_window):
    raise NotImplementedError("write your pallas kernel here")


# speedup vs baseline: 1.4919x; 1.4919x over previous
import functools

import jax
import jax.numpy as jnp
import numpy as np
from jax import lax
from jax.experimental import pallas as pl
from jax.experimental.pallas import tpu as pltpu
from jax.experimental.pallas import tpu_sc as plsc

_NC = 2
_NS = 16
_NW = _NC * _NS
_LANES = 16


def _make_sc_gather(num_signals, batch):
    b_per_w = batch // _NW
    mesh = plsc.VectorSubcoreMesh(core_axis_name="c", subcore_axis_name="s")

    @functools.partial(
        pl.kernel,
        mesh=mesh,
        out_type=(
            jax.ShapeDtypeStruct((batch, 64), jnp.float32),
            jax.ShapeDtypeStruct((batch, 512), jnp.float32),
            jax.ShapeDtypeStruct((batch, 16), jnp.float32),
        ),
        scratch_types=[
            pltpu.VMEM((b_per_w,), jnp.int32),
            pltpu.VMEM((b_per_w, 32), jnp.float32),
            pltpu.VMEM((b_per_w, 32), jnp.float32),
            pltpu.VMEM((b_per_w, 512), jnp.float32),
            pltpu.VMEM((b_per_w, 16), jnp.float32),
            pltpu.VMEM((b_per_w, 64), jnp.float32),
            pltpu.SemaphoreType.DMA,
            pltpu.SemaphoreType.DMA,
            pltpu.SemaphoreType.DMA,
        ],
        compiler_params=pltpu.CompilerParams(use_tc_tiling_on_sc=False),
    )
    def gather_kernel(idx_hbm, pp_hbm, po_hbm, a_hbm, gw_hbm,
                      p_out, a_out, gw_out,
                      idx_v, pp_v, po_v, a_v, gw_v, p_v,
                      sem_po, sem_a, sem_gw):
        wid = lax.axis_index("s") * _NC + lax.axis_index("c")
        base = wid * b_per_w

        pltpu.sync_copy(idx_hbm.at[pl.ds(base, b_per_w)], idx_v)

        cp_pp = pltpu.async_copy(pp_hbm.at[idx_v], pp_v, sem_po)
        cp_po = pltpu.async_copy(po_hbm.at[idx_v], po_v, sem_po)
        cp_a = pltpu.async_copy(a_hbm.at[idx_v], a_v, sem_a)
        cp_gw = pltpu.async_copy(gw_hbm.at[idx_v], gw_v, sem_gw)

        cp_pp.wait()
        cp_po.wait()

        lane = lax.broadcasted_iota(jnp.int32, (_LANES,), 0)

        def _splat(v):
            return jnp.full((_LANES,), v, dtype=jnp.int32)

        perm = [
            _splat(8 * (k % 2))
            + _splat(2) * lax.div(lane, _splat(4))
            + lax.rem(lane, _splat(2))
            for k in range(4)
        ]
        sel = lax.lt(lax.rem(lane, _splat(4)), _splat(2))

        gdn = lax.GatherDimensionNumbers(
            offset_dims=(), collapsed_slice_dims=(0,), start_index_map=(0,))

        def _permute(v, q):
            return lax.gather(
                v, q[:, None], gdn, (1,),
                mode=lax.GatherScatterMode.PROMISE_IN_BOUNDS)

        def body(j, _):
            for k in range(4):
                v_pp = pp_v[j, pl.ds(16 * (k // 2), 16)]
                v_po = po_v[j, pl.ds(16 * (k // 2), 16)]
                g_pp = _permute(v_pp, perm[k])
                g_po = _permute(v_po, perm[k])
                p_v[j, pl.ds(16 * k, 16)] = jnp.where(sel, g_pp, g_po)
            return 0

        lax.fori_loop(0, b_per_w, body, 0)

        pltpu.sync_copy(p_v, p_out.at[pl.ds(base, b_per_w)])
        cp_gw.wait()
        pltpu.sync_copy(gw_v, gw_out.at[pl.ds(base, b_per_w)])
        cp_a.wait()
        pltpu.sync_copy(a_v, a_out.at[pl.ds(base, b_per_w)])

    return gather_kernel


def kernel(idx, p_pos, p_ori, a, gaussian_window):
    num_signals, num_latents, pos_dims = p_pos.shape
    batch = idx.shape[0]
    latent_dim = a.shape[-1]
    ori_dims = p_ori.shape[-1]

    pp_flat = p_pos.reshape(num_signals, num_latents * pos_dims)
    po_flat = p_ori.reshape(num_signals, num_latents * ori_dims)
    a_flat = a.reshape(num_signals, num_latents * latent_dim)
    gw_flat = gaussian_window.reshape(num_signals, num_latents)

    fn = _make_sc_gather(num_signals, batch)
    p_f, a_f, gw_f = fn(idx, pp_flat, po_flat, a_flat, gw_flat)

    return (
        p_f.reshape(batch, num_latents, pos_dims + ori_dims),
        a_f.reshape(batch, num_latents, latent_dim),
        gw_f.reshape(batch, num_latents, 1),
    )
